# baseline (speedup 1.0000x reference)
import numpy as np
import jax
import jax.numpy as jnp
from jax import lax
from jax.experimental import pallas as pl
from jax.experimental.pallas import tpu as pltpu
from jax.experimental.pallas import tpu_sc as plsc

N = 10000
NP = 10240
R = 320
HW = 256
MW = 64
E = 320000
C = 1600
NCH = E // C
G = 48
M = 256
ALPHA = 0.7213 / (1.0 + 1.079 / M)
TH = 220.0

_lut = np.zeros(272, np.float32)
_z = np.arange(1, 257, dtype=np.float64)
_lut[1:257] = (M * np.log(M / _z)).astype(np.float32)


def _hop_body(src_hbm, dst_hbm, mh_in, hll_in, lut_hbm,
              mh_out, hll_out, cards_out,
              acc_h, acc_m, srcb_a, dstb_a, srcb_b, dstb_b,
              msrc, mdst, pbuf, hrows, mrows,
              lutv, zscr, sescr, cardsv, sem1, sem2, sem_a, sem_b):
    wid = lax.axis_index("s") * 2 + lax.axis_index("c")
    base = wid * R

    pltpu.async_copy(src_hbm.at[pl.ds(0, C)], srcb_a, sem_a)
    pltpu.async_copy(dst_hbm.at[pl.ds(0, C)], dstb_a, sem_a)

    pltpu.sync_copy(hll_in.at[pl.ds(base, R)], acc_h)
    pltpu.sync_copy(mh_in.at[pl.ds(base, R)], acc_m)
    pltpu.sync_copy(lut_hbm, lutv)

    def zi(i, carry):
        msrc[pl.ds(i * 16, 16)] = jnp.zeros((16,), jnp.int32)
        return carry
    lax.fori_loop(0, (C + 16) // 16, zi, 0)
    pbuf[pl.ds(0, 16)] = jnp.zeros((16,), jnp.int32)
    pbuf[pl.ds(16, 16)] = jnp.zeros((16,), jnp.int32)

    def process(srcb, dstb):
        def scan_body(i, off):
            sv = srcb[pl.ds(i * 16, 16)]
            dv = dstb[pl.ds(i * 16, 16)]
            m = (dv >= base) & (dv < base + R)
            mi = jnp.where(m, jnp.int32(1), jnp.int32(0))
            pbuf[pl.ds(16, 16)] = mi
            x = mi + pbuf[pl.ds(15, 16)]
            pbuf[pl.ds(16, 16)] = x
            x = x + pbuf[pl.ds(14, 16)]
            pbuf[pl.ds(16, 16)] = x
            x = x + pbuf[pl.ds(12, 16)]
            pbuf[pl.ds(16, 16)] = x
            x = x + pbuf[pl.ds(8, 16)]
            pos = off + x - 1
            plsc.store_scatter(msrc, [pos], sv, mask=m)
            plsc.store_scatter(mdst, [pos], dv - base, mask=m)
            return off + x[15]
        cnt = lax.fori_loop(0, C // 16, scan_body, jnp.int32(0))

        ng = (cnt + (G - 1)) // G

        def group_body(g, carry):
            gbase = g * G
            cp1 = pltpu.async_copy(
                hll_in.at[msrc.at[pl.ds(gbase, G)]], hrows, sem1)
            cp2 = pltpu.async_copy(
                mh_in.at[msrc.at[pl.ds(gbase, G)]], mrows, sem2)
            cp1.wait()
            cp2.wait()
            ne = jnp.minimum(G, cnt - gbase)

            def edge_body(j, carry2):
                dl = mdst[pl.ds(gbase + j, 16)][0]
                for k in range(HW // 16):
                    s = pl.ds(k * 16, 16)
                    acc_h[dl, s] = jnp.maximum(acc_h[dl, s], hrows[j, s])
                for k in range(MW // 16):
                    s = pl.ds(k * 16, 16)
                    acc_m[dl, s] = jnp.minimum(acc_m[dl, s], mrows[j, s])
                return carry2
            lax.fori_loop(0, ne, edge_body, 0)
            return carry
        lax.fori_loop(0, ng, group_body, 0)

    def chunk2_body(h, carry):
        c0 = 2 * h
        pltpu.make_async_copy(src_hbm.at[pl.ds(0, C)], srcb_a, sem_a).wait()
        pltpu.make_async_copy(dst_hbm.at[pl.ds(0, C)], dstb_a, sem_a).wait()
        pltpu.async_copy(src_hbm.at[pl.ds((c0 + 1) * C, C)], srcb_b, sem_b)
        pltpu.async_copy(dst_hbm.at[pl.ds((c0 + 1) * C, C)], dstb_b, sem_b)
        process(srcb_a, dstb_a)
        pltpu.make_async_copy(src_hbm.at[pl.ds(0, C)], srcb_b, sem_b).wait()
        pltpu.make_async_copy(dst_hbm.at[pl.ds(0, C)], dstb_b, sem_b).wait()

        @pl.when(h + 1 < NCH // 2)
        def _prefetch():
            pltpu.async_copy(
                src_hbm.at[pl.ds((c0 + 2) * C, C)], srcb_a, sem_a)
            pltpu.async_copy(
                dst_hbm.at[pl.ds((c0 + 2) * C, C)], dstb_a, sem_a)
        process(srcb_b, dstb_b)
        return carry
    lax.fori_loop(0, NCH // 2, chunk2_body, 0)

    lane = lax.iota(jnp.int32, 16)

    def rg_body(rg, carry):
        def row_body(rl, carry2):
            row = rg * 16 + rl
            zp = jnp.zeros((16,), jnp.int32)
            sp = jnp.zeros((16,), jnp.float32)
            for k in range(HW // 16):
                v = acc_h[row, pl.ds(k * 16, 16)]
                zp = zp + jnp.where(v == 0.0, jnp.int32(1), jnp.int32(0))
                vi = v.astype(jnp.int32)
                bits = (127 << 23) - (vi << 23)
                sp = sp + lax.bitcast_convert_type(bits, jnp.float32)
            idx = lane * 16 + rl
            plsc.store_scatter(zscr, [idx], zp)
            plsc.store_scatter(sescr, [idx], sp)
            return carry2
        lax.fori_loop(0, 16, row_body, 0)
        nz = zscr[pl.ds(0, 16)]
        se = sescr[pl.ds(0, 16)]
        for l in range(1, 16):
            nz = nz + zscr[pl.ds(l * 16, 16)]
            se = se + sescr[pl.ds(l * 16, 16)]
        lc = plsc.load_gather(lutv, [nz])
        card = jnp.where(nz > 0, lc, TH + 1.0)
        card = jnp.where(card > TH, (ALPHA * M * M) / se, card)
        cardsv[pl.ds(rg * 16, 16)] = card
        return carry
    lax.fori_loop(0, R // 16, rg_body, 0)

    pltpu.sync_copy(acc_h, hll_out.at[pl.ds(base, R)])
    pltpu.sync_copy(acc_m, mh_out.at[pl.ds(base, R)])
    pltpu.sync_copy(cardsv, cards_out.at[pl.ds(base, R)])


def _make_hop():
    mesh = plsc.VectorSubcoreMesh(core_axis_name="c", subcore_axis_name="s")
    return pl.kernel(
        _hop_body,
        out_type=(
            jax.ShapeDtypeStruct((NP, MW), jnp.int32),
            jax.ShapeDtypeStruct((NP, HW), jnp.float32),
            jax.ShapeDtypeStruct((NP,), jnp.float32),
        ),
        mesh=mesh,
        compiler_params=pltpu.CompilerParams(
            needs_layout_passes=False, use_tc_tiling_on_sc=False),
        scratch_types=[
            pltpu.VMEM((R, HW), jnp.float32),
            pltpu.VMEM((R, MW), jnp.int32),
            pltpu.VMEM((C,), jnp.int32),
            pltpu.VMEM((C,), jnp.int32),
            pltpu.VMEM((C,), jnp.int32),
            pltpu.VMEM((C,), jnp.int32),
            pltpu.VMEM((C + 16,), jnp.int32),
            pltpu.VMEM((C + 16,), jnp.int32),
            pltpu.VMEM((32,), jnp.int32),
            pltpu.VMEM((G, HW), jnp.float32),
            pltpu.VMEM((G, MW), jnp.int32),
            pltpu.VMEM((272,), jnp.float32),
            pltpu.VMEM((256,), jnp.int32),
            pltpu.VMEM((256,), jnp.float32),
            pltpu.VMEM((R,), jnp.float32),
            pltpu.SemaphoreType.DMA,
            pltpu.SemaphoreType.DMA,
            pltpu.SemaphoreType.DMA,
            pltpu.SemaphoreType.DMA,
        ],
        name="elph_hop",
    )


def kernel(minhash, hll_regs, edge_index):
    src = edge_index[0]
    dst = edge_index[1]
    mh_p = jnp.pad(minhash, ((0, NP - N), (0, 0)))
    hll_p = jnp.pad(hll_regs, ((0, NP - N), (0, 0)))
    lut = jnp.asarray(_lut)
    hop = _make_hop()
    mh1, hll1, c1 = hop(src, dst, mh_p, hll_p, lut)
    mh2, hll2, c2 = hop(src, dst, mh1, hll1, lut)
    cards = jnp.stack([c1[:N], c2[:N]], axis=1)
    return (mh2[:N], hll2[:N], cards)


# speedup vs baseline: 2.1259x; 2.1259x over previous
import numpy as np
import jax
import jax.numpy as jnp
from jax import lax
from jax.experimental import pallas as pl
from jax.experimental.pallas import tpu as pltpu
from jax.experimental.pallas import tpu_sc as plsc

N = 10000
NP = 10240
R = 320
HW = 256
MW = 64
E = 320000
C = 1600
NCH = E // C
G = 64
M = 256
ALPHA = 0.7213 / (1.0 + 1.079 / M)
TH = 220.0

_lut = np.zeros(272, np.float32)
_z = np.arange(1, 257, dtype=np.float64)
_lut[1:257] = (M * np.log(M / _z)).astype(np.float32)


def _hop_body(src_hbm, dst_hbm, mh_in, hll_in, lut_hbm,
              mh_out, hll_out, cards_out,
              acc_h, acc_m, srcb, dstb,
              msrc, mdst, pbuf, hrows, mrows,
              lutv, zscr, sescr, cardsv, sem1, sem2):
    wid = lax.axis_index("s") * 2 + lax.axis_index("c")
    base = wid * R

    pltpu.sync_copy(hll_in.at[pl.ds(base, R)], acc_h)
    pltpu.sync_copy(mh_in.at[pl.ds(base, R)], acc_m)
    pltpu.sync_copy(lut_hbm, lutv)

    def zi(i, carry):
        msrc[pl.ds(i * 16, 16)] = jnp.zeros((16,), jnp.int32)
        return carry
    lax.fori_loop(0, (C + 16) // 16, zi, 0)
    pbuf[pl.ds(0, 16)] = jnp.zeros((16,), jnp.int32)
    pbuf[pl.ds(16, 16)] = jnp.zeros((16,), jnp.int32)

    def chunk_body(c, carry):
        pltpu.sync_copy(src_hbm.at[pl.ds(c * C, C)], srcb)
        pltpu.sync_copy(dst_hbm.at[pl.ds(c * C, C)], dstb)

        def scan_body(i, off):
            sv = srcb[pl.ds(i * 16, 16)]
            dv = dstb[pl.ds(i * 16, 16)]
            m = (dv >= base) & (dv < base + R)
            mi = jnp.where(m, jnp.int32(1), jnp.int32(0))
            pbuf[pl.ds(16, 16)] = mi
            x = mi + pbuf[pl.ds(15, 16)]
            pbuf[pl.ds(16, 16)] = x
            x = x + pbuf[pl.ds(14, 16)]
            pbuf[pl.ds(16, 16)] = x
            x = x + pbuf[pl.ds(12, 16)]
            pbuf[pl.ds(16, 16)] = x
            x = x + pbuf[pl.ds(8, 16)]
            pos = off + x - 1
            plsc.store_scatter(msrc, [pos], sv, mask=m)
            plsc.store_scatter(mdst, [pos], dv - base, mask=m)
            return off + x[15]
        cnt = lax.fori_loop(0, C // 16, scan_body, jnp.int32(0))

        ng = (cnt + (G - 1)) // G

        def group_body(g, carry):
            gbase = g * G
            cp1 = pltpu.async_copy(
                hll_in.at[msrc.at[pl.ds(gbase, G)]], hrows, sem1)
            cp2 = pltpu.async_copy(
                mh_in.at[msrc.at[pl.ds(gbase, G)]], mrows, sem2)
            cp1.wait()
            cp2.wait()
            ne = jnp.minimum(G, cnt - gbase)

            def edge_body(j, carry2):
                dl = mdst[pl.ds(gbase + j, 16)][0]
                for k in range(HW // 16):
                    s = pl.ds(k * 16, 16)
                    acc_h[dl, s] = jnp.maximum(acc_h[dl, s], hrows[j, s])
                for k in range(MW // 16):
                    s = pl.ds(k * 16, 16)
                    acc_m[dl, s] = jnp.minimum(acc_m[dl, s], mrows[j, s])
                return carry2
            lax.fori_loop(0, ne, edge_body, 0)
            return carry
        lax.fori_loop(0, ng, group_body, 0)
        return carry
    lax.fori_loop(0, NCH, chunk_body, 0)

    lane = lax.iota(jnp.int32, 16)

    def rg_body(rg, carry):
        def row_body(rl, carry2):
            row = rg * 16 + rl
            zp = jnp.zeros((16,), jnp.int32)
            sp = jnp.zeros((16,), jnp.float32)
            for k in range(HW // 16):
                v = acc_h[row, pl.ds(k * 16, 16)]
                zp = zp + jnp.where(v == 0.0, jnp.int32(1), jnp.int32(0))
                vi = v.astype(jnp.int32)
                bits = (127 << 23) - (vi << 23)
                sp = sp + lax.bitcast_convert_type(bits, jnp.float32)
            idx = lane * 16 + rl
            plsc.store_scatter(zscr, [idx], zp)
            plsc.store_scatter(sescr, [idx], sp)
            return carry2
        lax.fori_loop(0, 16, row_body, 0)
        nz = zscr[pl.ds(0, 16)]
        se = sescr[pl.ds(0, 16)]
        for l in range(1, 16):
            nz = nz + zscr[pl.ds(l * 16, 16)]
            se = se + sescr[pl.ds(l * 16, 16)]
        lc = plsc.load_gather(lutv, [nz])
        card = jnp.where(nz > 0, lc, TH + 1.0)
        card = jnp.where(card > TH, (ALPHA * M * M) / se, card)
        cardsv[pl.ds(rg * 16, 16)] = card
        return carry
    lax.fori_loop(0, R // 16, rg_body, 0)

    pltpu.sync_copy(acc_h, hll_out.at[pl.ds(base, R)])
    pltpu.sync_copy(acc_m, mh_out.at[pl.ds(base, R)])
    pltpu.sync_copy(cardsv, cards_out.at[pl.ds(base, R)])


def _make_hop():
    mesh = plsc.VectorSubcoreMesh(core_axis_name="c", subcore_axis_name="s")
    return pl.kernel(
        _hop_body,
        out_type=(
            jax.ShapeDtypeStruct((NP, MW), jnp.int32),
            jax.ShapeDtypeStruct((NP, HW), jnp.float32),
            jax.ShapeDtypeStruct((NP,), jnp.float32),
        ),
        mesh=mesh,
        compiler_params=pltpu.CompilerParams(
            needs_layout_passes=False, use_tc_tiling_on_sc=False),
        scratch_types=[
            pltpu.VMEM((R, HW), jnp.float32),
            pltpu.VMEM((R, MW), jnp.int32),
            pltpu.VMEM((C,), jnp.int32),
            pltpu.VMEM((C,), jnp.int32),
            pltpu.VMEM((C + 16,), jnp.int32),
            pltpu.VMEM((C + 16,), jnp.int32),
            pltpu.VMEM((32,), jnp.int32),
            pltpu.VMEM((G, HW), jnp.float32),
            pltpu.VMEM((G, MW), jnp.int32),
            pltpu.VMEM((272,), jnp.float32),
            pltpu.VMEM((256,), jnp.int32),
            pltpu.VMEM((256,), jnp.float32),
            pltpu.VMEM((R,), jnp.float32),
            pltpu.SemaphoreType.DMA,
            pltpu.SemaphoreType.DMA,
        ],
        name="elph_hop",
    )


def kernel(minhash, hll_regs, edge_index):
    src = edge_index[0]
    dst = edge_index[1]
    mh_p = jnp.pad(minhash, ((0, NP - N), (0, 0)))
    hll_p = jnp.pad(hll_regs, ((0, NP - N), (0, 0)))
    lut = jnp.asarray(_lut)
    hop = _make_hop()
    mh1, hll1, c1 = hop(src, dst, mh_p, hll_p, lut)
    mh2, hll2, c2 = hop(src, dst, mh1, hll1, lut)
    cards = jnp.stack([c1[:N], c2[:N]], axis=1)
    return (mh2[:N], hll2[:N], cards)


# speedup vs baseline: 2.2333x; 1.0505x over previous
import numpy as np
import jax
import jax.numpy as jnp
from jax import lax
from jax.experimental import pallas as pl
from jax.experimental.pallas import tpu as pltpu
from jax.experimental.pallas import tpu_sc as plsc

N = 10000
NP = 10240
R = 320
HW = 256
MW = 64
E = 320000
C = 1600
NCH = E // C
G = 64
M = 256
ALPHA = 0.7213 / (1.0 + 1.079 / M)
TH = 220.0

_lut = np.zeros(272, np.float32)
_z = np.arange(1, 257, dtype=np.float64)
_lut[1:257] = (M * np.log(M / _z)).astype(np.float32)


def _hop_body(src_hbm, dst_hbm, mh_in, hll_in, lut_hbm,
              mh_out, hll_out, cards_out,
              acc_h, acc_m, srcb, dstb,
              msrc, mdst, pbuf, hrows, mrows,
              lutv, zscr, sescr, cardsv, sem1, sem2):
    wid = lax.axis_index("s") * 2 + lax.axis_index("c")
    base = wid * R

    pltpu.sync_copy(hll_in.at[pl.ds(base, R)], acc_h)
    pltpu.sync_copy(mh_in.at[pl.ds(base, R)], acc_m)
    pltpu.sync_copy(lut_hbm, lutv)

    def zi(i, carry):
        msrc[pl.ds(i * 16, 16)] = jnp.zeros((16,), jnp.int32)
        return carry
    lax.fori_loop(0, (C + 16) // 16, zi, 0)
    pbuf[pl.ds(0, 16)] = jnp.zeros((16,), jnp.int32)
    pbuf[pl.ds(16, 16)] = jnp.zeros((16,), jnp.int32)

    def chunk_body(c, carry):
        cpe1 = pltpu.async_copy(src_hbm.at[pl.ds(c * C, C)], srcb, sem1)
        cpe2 = pltpu.async_copy(dst_hbm.at[pl.ds(c * C, C)], dstb, sem2)
        cpe1.wait()
        cpe2.wait()

        def scan_body(i, off):
            sv = srcb[pl.ds(i * 16, 16)]
            dv = dstb[pl.ds(i * 16, 16)]
            m = (dv >= base) & (dv < base + R)
            mi = jnp.where(m, jnp.int32(1), jnp.int32(0))
            pbuf[pl.ds(16, 16)] = mi
            x = mi + pbuf[pl.ds(15, 16)]
            pbuf[pl.ds(16, 16)] = x
            x = x + pbuf[pl.ds(14, 16)]
            pbuf[pl.ds(16, 16)] = x
            x = x + pbuf[pl.ds(12, 16)]
            pbuf[pl.ds(16, 16)] = x
            x = x + pbuf[pl.ds(8, 16)]
            pos = off + x - 1
            plsc.store_scatter(msrc, [pos], sv, mask=m)
            plsc.store_scatter(mdst, [pos], dv - base, mask=m)
            return off + x[15]
        cnt = lax.fori_loop(0, C // 16, scan_body, jnp.int32(0))

        ng = (cnt + (G - 1)) // G

        def group_body(g, carry):
            gbase = g * G
            cp1 = pltpu.async_copy(
                hll_in.at[msrc.at[pl.ds(gbase, G)]], hrows, sem1)
            cp2 = pltpu.async_copy(
                mh_in.at[msrc.at[pl.ds(gbase, G)]], mrows, sem2)
            cp1.wait()
            cp2.wait()
            ne = jnp.minimum(G, cnt - gbase)

            def edge_body(j, carry2):
                dl = mdst[pl.ds(gbase + j, 16)][0]
                for k in range(HW // 16):
                    s = pl.ds(k * 16, 16)
                    acc_h[dl, s] = jnp.maximum(acc_h[dl, s], hrows[j, s])
                for k in range(MW // 16):
                    s = pl.ds(k * 16, 16)
                    acc_m[dl, s] = jnp.minimum(acc_m[dl, s], mrows[j, s])
                return carry2
            lax.fori_loop(0, ne, edge_body, 0)
            return carry
        lax.fori_loop(0, ng, group_body, 0)
        return carry
    lax.fori_loop(0, NCH, chunk_body, 0)

    lane = lax.iota(jnp.int32, 16)

    def rg_body(rg, carry):
        def row_body(rl, carry2):
            row = rg * 16 + rl
            zp = jnp.zeros((16,), jnp.int32)
            sp = jnp.zeros((16,), jnp.float32)
            for k in range(HW // 16):
                v = acc_h[row, pl.ds(k * 16, 16)]
                zp = zp + jnp.where(v == 0.0, jnp.int32(1), jnp.int32(0))
                vi = v.astype(jnp.int32)
                bits = (127 << 23) - (vi << 23)
                sp = sp + lax.bitcast_convert_type(bits, jnp.float32)
            idx = lane * 16 + rl
            plsc.store_scatter(zscr, [idx], zp)
            plsc.store_scatter(sescr, [idx], sp)
            return carry2
        lax.fori_loop(0, 16, row_body, 0)
        nz = zscr[pl.ds(0, 16)]
        se = sescr[pl.ds(0, 16)]
        for l in range(1, 16):
            nz = nz + zscr[pl.ds(l * 16, 16)]
            se = se + sescr[pl.ds(l * 16, 16)]
        lc = plsc.load_gather(lutv, [nz])
        card = jnp.where(nz > 0, lc, TH + 1.0)
        card = jnp.where(card > TH, (ALPHA * M * M) / se, card)
        cardsv[pl.ds(rg * 16, 16)] = card
        return carry
    lax.fori_loop(0, R // 16, rg_body, 0)

    pltpu.sync_copy(acc_h, hll_out.at[pl.ds(base, R)])
    pltpu.sync_copy(acc_m, mh_out.at[pl.ds(base, R)])
    pltpu.sync_copy(cardsv, cards_out.at[pl.ds(base, R)])


def _make_hop():
    mesh = plsc.VectorSubcoreMesh(core_axis_name="c", subcore_axis_name="s")
    return pl.kernel(
        _hop_body,
        out_type=(
            jax.ShapeDtypeStruct((NP, MW), jnp.int32),
            jax.ShapeDtypeStruct((NP, HW), jnp.float32),
            jax.ShapeDtypeStruct((NP,), jnp.float32),
        ),
        mesh=mesh,
        compiler_params=pltpu.CompilerParams(
            needs_layout_passes=False, use_tc_tiling_on_sc=False),
        scratch_types=[
            pltpu.VMEM((R, HW), jnp.float32),
            pltpu.VMEM((R, MW), jnp.int32),
            pltpu.VMEM((C,), jnp.int32),
            pltpu.VMEM((C,), jnp.int32),
            pltpu.VMEM((C + 16,), jnp.int32),
            pltpu.VMEM((C + 16,), jnp.int32),
            pltpu.VMEM((32,), jnp.int32),
            pltpu.VMEM((G, HW), jnp.float32),
            pltpu.VMEM((G, MW), jnp.int32),
            pltpu.VMEM((272,), jnp.float32),
            pltpu.VMEM((256,), jnp.int32),
            pltpu.VMEM((256,), jnp.float32),
            pltpu.VMEM((R,), jnp.float32),
            pltpu.SemaphoreType.DMA,
            pltpu.SemaphoreType.DMA,
        ],
        name="elph_hop",
    )


def kernel(minhash, hll_regs, edge_index):
    src = edge_index[0]
    dst = edge_index[1]
    mh_p = jnp.pad(minhash, ((0, NP - N), (0, 0)))
    hll_p = jnp.pad(hll_regs, ((0, NP - N), (0, 0)))
    lut = jnp.asarray(_lut)
    hop = _make_hop()
    mh1, hll1, c1 = hop(src, dst, mh_p, hll_p, lut)
    mh2, hll2, c2 = hop(src, dst, mh1, hll1, lut)
    cards = jnp.stack([c1[:N], c2[:N]], axis=1)
    return (mh2[:N], hll2[:N], cards)


# speedup vs baseline: 2.3701x; 1.0613x over previous
import numpy as np
import jax
import jax.numpy as jnp
from jax import lax
from jax.experimental import pallas as pl
from jax.experimental.pallas import tpu as pltpu
from jax.experimental.pallas import tpu_sc as plsc

N = 10000
NP = 10240
R = 320
HW = 256
MW = 64
E = 320000
C = 1600
NCH = E // C
G = 64
M = 256
ALPHA = 0.7213 / (1.0 + 1.079 / M)
TH = 220.0

_lut = np.zeros(272, np.float32)
_z = np.arange(1, 257, dtype=np.float64)
_lut[1:257] = (M * np.log(M / _z)).astype(np.float32)


def _hop_body(src_hbm, dst_hbm, mh_in, hll_in, lut_hbm,
              mh_out, hll_out, cards_out,
              acc_h, acc_m, srcb, dstb,
              msrc, mdst, pbuf, hrows, mrows,
              lutv, zscr, sescr, cardsv, sem1, sem2, sem3, sem4):
    wid = lax.axis_index("s") * 2 + lax.axis_index("c")
    base = wid * R

    pltpu.sync_copy(hll_in.at[pl.ds(base, R)], acc_h)
    pltpu.sync_copy(mh_in.at[pl.ds(base, R)], acc_m)
    pltpu.sync_copy(lut_hbm, lutv)

    def zi(i, carry):
        msrc[pl.ds(i * 16, 16)] = jnp.zeros((16,), jnp.int32)
        return carry
    lax.fori_loop(0, (C + 16) // 16, zi, 0)
    pbuf[pl.ds(0, 16)] = jnp.zeros((16,), jnp.int32)
    pbuf[pl.ds(16, 16)] = jnp.zeros((16,), jnp.int32)

    pltpu.async_copy(src_hbm.at[pl.ds(0, C)], srcb, sem3)
    pltpu.async_copy(dst_hbm.at[pl.ds(0, C)], dstb, sem4)

    def chunk_body(c, carry):
        pltpu.make_async_copy(src_hbm.at[pl.ds(0, C)], srcb, sem3).wait()
        pltpu.make_async_copy(dst_hbm.at[pl.ds(0, C)], dstb, sem4).wait()

        def scan_body(i, off):
            sv = srcb[pl.ds(i * 16, 16)]
            dv = dstb[pl.ds(i * 16, 16)]
            m = (dv >= base) & (dv < base + R)
            mi = jnp.where(m, jnp.int32(1), jnp.int32(0))
            pbuf[pl.ds(16, 16)] = mi
            x = mi + pbuf[pl.ds(15, 16)]
            pbuf[pl.ds(16, 16)] = x
            x = x + pbuf[pl.ds(14, 16)]
            pbuf[pl.ds(16, 16)] = x
            x = x + pbuf[pl.ds(12, 16)]
            pbuf[pl.ds(16, 16)] = x
            x = x + pbuf[pl.ds(8, 16)]
            pos = off + x - 1
            plsc.store_scatter(msrc, [pos], sv, mask=m)
            plsc.store_scatter(mdst, [pos], dv - base, mask=m)
            return off + x[15]
        cnt = lax.fori_loop(0, C // 16, scan_body, jnp.int32(0))

        cn = jnp.minimum(c + 1, NCH - 1) * C
        pltpu.async_copy(src_hbm.at[pl.ds(cn, C)], srcb, sem3)
        pltpu.async_copy(dst_hbm.at[pl.ds(cn, C)], dstb, sem4)

        ng = (cnt + (G - 1)) // G

        def group_body(g, carry):
            gbase = g * G
            cp1 = pltpu.async_copy(
                hll_in.at[msrc.at[pl.ds(gbase, G)]], hrows, sem1)
            cp2 = pltpu.async_copy(
                mh_in.at[msrc.at[pl.ds(gbase, G)]], mrows, sem2)
            cp1.wait()
            cp2.wait()
            ne = jnp.minimum(G, cnt - gbase)

            def edge_body(j, carry2):
                dl = mdst[pl.ds(gbase + j, 16)][0]
                for k in range(HW // 16):
                    s = pl.ds(k * 16, 16)
                    acc_h[dl, s] = jnp.maximum(acc_h[dl, s], hrows[j, s])
                for k in range(MW // 16):
                    s = pl.ds(k * 16, 16)
                    acc_m[dl, s] = jnp.minimum(acc_m[dl, s], mrows[j, s])
                return carry2
            lax.fori_loop(0, ne, edge_body, 0)
            return carry
        lax.fori_loop(0, ng, group_body, 0)
        return carry
    lax.fori_loop(0, NCH, chunk_body, 0)
    pltpu.make_async_copy(src_hbm.at[pl.ds(0, C)], srcb, sem3).wait()
    pltpu.make_async_copy(dst_hbm.at[pl.ds(0, C)], dstb, sem4).wait()

    lane = lax.iota(jnp.int32, 16)

    def rg_body(rg, carry):
        def row_body(rl, carry2):
            row = rg * 16 + rl
            zp = jnp.zeros((16,), jnp.int32)
            sp = jnp.zeros((16,), jnp.float32)
            for k in range(HW // 16):
                v = acc_h[row, pl.ds(k * 16, 16)]
                zp = zp + jnp.where(v == 0.0, jnp.int32(1), jnp.int32(0))
                vi = v.astype(jnp.int32)
                bits = (127 << 23) - (vi << 23)
                sp = sp + lax.bitcast_convert_type(bits, jnp.float32)
            idx = lane * 16 + rl
            plsc.store_scatter(zscr, [idx], zp)
            plsc.store_scatter(sescr, [idx], sp)
            return carry2
        lax.fori_loop(0, 16, row_body, 0)
        nz = zscr[pl.ds(0, 16)]
        se = sescr[pl.ds(0, 16)]
        for l in range(1, 16):
            nz = nz + zscr[pl.ds(l * 16, 16)]
            se = se + sescr[pl.ds(l * 16, 16)]
        lc = plsc.load_gather(lutv, [nz])
        card = jnp.where(nz > 0, lc, TH + 1.0)
        card = jnp.where(card > TH, (ALPHA * M * M) / se, card)
        cardsv[pl.ds(rg * 16, 16)] = card
        return carry
    lax.fori_loop(0, R // 16, rg_body, 0)

    pltpu.sync_copy(acc_h, hll_out.at[pl.ds(base, R)])
    pltpu.sync_copy(acc_m, mh_out.at[pl.ds(base, R)])
    pltpu.sync_copy(cardsv, cards_out.at[pl.ds(base, R)])


def _make_hop():
    mesh = plsc.VectorSubcoreMesh(core_axis_name="c", subcore_axis_name="s")
    return pl.kernel(
        _hop_body,
        out_type=(
            jax.ShapeDtypeStruct((NP, MW), jnp.int32),
            jax.ShapeDtypeStruct((NP, HW), jnp.float32),
            jax.ShapeDtypeStruct((NP,), jnp.float32),
        ),
        mesh=mesh,
        compiler_params=pltpu.CompilerParams(
            needs_layout_passes=False, use_tc_tiling_on_sc=False),
        scratch_types=[
            pltpu.VMEM((R, HW), jnp.float32),
            pltpu.VMEM((R, MW), jnp.int32),
            pltpu.VMEM((C,), jnp.int32),
            pltpu.VMEM((C,), jnp.int32),
            pltpu.VMEM((C + 16,), jnp.int32),
            pltpu.VMEM((C + 16,), jnp.int32),
            pltpu.VMEM((32,), jnp.int32),
            pltpu.VMEM((G, HW), jnp.float32),
            pltpu.VMEM((G, MW), jnp.int32),
            pltpu.VMEM((272,), jnp.float32),
            pltpu.VMEM((256,), jnp.int32),
            pltpu.VMEM((256,), jnp.float32),
            pltpu.VMEM((R,), jnp.float32),
            pltpu.SemaphoreType.DMA,
            pltpu.SemaphoreType.DMA,
            pltpu.SemaphoreType.DMA,
            pltpu.SemaphoreType.DMA,
        ],
        name="elph_hop",
    )


def kernel(minhash, hll_regs, edge_index):
    src = edge_index[0]
    dst = edge_index[1]
    mh_p = jnp.pad(minhash, ((0, NP - N), (0, 0)))
    hll_p = jnp.pad(hll_regs, ((0, NP - N), (0, 0)))
    lut = jnp.asarray(_lut)
    hop = _make_hop()
    mh1, hll1, c1 = hop(src, dst, mh_p, hll_p, lut)
    mh2, hll2, c2 = hop(src, dst, mh1, hll1, lut)
    cards = jnp.stack([c1[:N], c2[:N]], axis=1)
    return (mh2[:N], hll2[:N], cards)
